# baseline (speedup 1.0000x reference)
import functools

import jax
import jax.numpy as jnp
from jax import lax
from jax.experimental import pallas as pl
from jax.experimental.pallas import tpu as pltpu
from jax.experimental.pallas import tpu_sc as plsc

NC, NS = 2, 16
NW = NC * NS
K = 8
NBUF = 4
LAG = 3


def _make_gather(n_ids: int, hidden: int):
  bpw = n_ids // NW
  nch = bpw // K
  mesh = plsc.VectorSubcoreMesh(core_axis_name="c", subcore_axis_name="s")

  @functools.partial(
      pl.kernel,
      mesh=mesh,
      out_type=jax.ShapeDtypeStruct((n_ids, hidden), jnp.float32),
      scratch_types=[
          pltpu.VMEM((bpw,), jnp.int32),
          [pltpu.VMEM((K, hidden), jnp.float32) for _ in range(NBUF)],
          [pltpu.SemaphoreType.DMA for _ in range(NBUF)],
          [pltpu.SemaphoreType.DMA for _ in range(NBUF)],
      ],
  )
  def gather(tbl_hbm, idx_hbm, out_hbm, idx_v, bufs, gsems, ssems):
    wid = lax.axis_index("s") * NC + lax.axis_index("c")
    base = pl.multiple_of(wid * bpw, 8)
    pltpu.sync_copy(idx_hbm.at[pl.ds(base, bpw)], idx_v)

    def g_desc(ch, j):
      off = pl.multiple_of(ch * K, 8)
      return pltpu.make_async_copy(
          tbl_hbm.at[idx_v.at[pl.ds(off, K)]], bufs[j], gsems[j])

    def s_desc(ch, j):
      off = pl.multiple_of(ch * K, 8)
      return pltpu.make_async_copy(
          bufs[j], out_hbm.at[pl.ds(base + off, K)], ssems[j])

    for j in range(NBUF - LAG):
      g_desc(j, j).start()

    @pl.loop(0, nch, step=NBUF)
    def _(t):
      for j in range(NBUF):
        ch = t + j
        g_desc(ch, j).wait()
        s_desc(ch, j).start()
        jn = (j + NBUF - LAG) % NBUF

        @pl.when(ch + NBUF - LAG < nch)
        def _():
          @pl.when(ch + NBUF - LAG >= NBUF)
          def _():
            s_desc(ch - LAG, jn).wait()
          g_desc(ch + NBUF - LAG, jn).start()

    for c in range(nch - NBUF, nch):
      s_desc(c, c % NBUF).wait()

  return gather


@jax.jit
def kernel(input_ids, embed_table):
  b, s = input_ids.shape
  v, h = embed_table.shape
  ids = input_ids.reshape(b * s).astype(jnp.int32)
  out = _make_gather(b * s, h)(embed_table, ids)
  return out.reshape(b, s, h)


# speedup vs baseline: 1.2330x; 1.2330x over previous
import functools

import jax
import jax.numpy as jnp
from jax import lax
from jax.experimental import pallas as pl
from jax.experimental.pallas import tpu as pltpu
from jax.experimental.pallas import tpu_sc as plsc

NC, NS = 2, 16
NW = NC * NS
K = 8
NBUF = 4
LAG = 1


def _make_gather(n_ids: int, hidden: int):
  bpw = n_ids // NW
  nch = bpw // K
  mesh = plsc.VectorSubcoreMesh(core_axis_name="c", subcore_axis_name="s")

  @functools.partial(
      pl.kernel,
      mesh=mesh,
      out_type=jax.ShapeDtypeStruct((n_ids, hidden), jnp.float32),
      scratch_types=[
          pltpu.VMEM((bpw,), jnp.int32),
          [pltpu.VMEM((K, hidden), jnp.float32) for _ in range(NBUF)],
          [pltpu.SemaphoreType.DMA for _ in range(NBUF)],
          [pltpu.SemaphoreType.DMA for _ in range(NBUF)],
      ],
  )
  def gather(tbl_hbm, idx_hbm, out_hbm, idx_v, bufs, gsems, ssems):
    wid = lax.axis_index("s") * NC + lax.axis_index("c")
    base = pl.multiple_of(wid * bpw, 8)
    pltpu.sync_copy(idx_hbm.at[pl.ds(base, bpw)], idx_v)

    def g_desc(ch, j):
      off = pl.multiple_of(ch * K, 8)
      return pltpu.make_async_copy(
          tbl_hbm.at[idx_v.at[pl.ds(off, K)]], bufs[j], gsems[j])

    def s_desc(ch, j):
      off = pl.multiple_of(ch * K, 8)
      return pltpu.make_async_copy(
          bufs[j], out_hbm.at[pl.ds(base + off, K)], ssems[j])

    for j in range(NBUF - LAG):
      g_desc(j, j).start()

    @pl.loop(0, nch, step=NBUF)
    def _(t):
      for j in range(NBUF):
        ch = t + j
        g_desc(ch, j).wait()
        s_desc(ch, j).start()
        jn = (j + NBUF - LAG) % NBUF

        @pl.when(ch + NBUF - LAG < nch)
        def _():
          @pl.when(ch + NBUF - LAG >= NBUF)
          def _():
            s_desc(ch - LAG, jn).wait()
          g_desc(ch + NBUF - LAG, jn).start()

    for c in range(nch - NBUF, nch):
      s_desc(c, c % NBUF).wait()

  return gather


@jax.jit
def kernel(input_ids, embed_table):
  b, s = input_ids.shape
  v, h = embed_table.shape
  ids = input_ids.reshape(b * s).astype(jnp.int32)
  out = _make_gather(b * s, h)(embed_table, ids)
  return out.reshape(b, s, h)


# speedup vs baseline: 1.2362x; 1.0025x over previous
import functools

import jax
import jax.numpy as jnp
from jax import lax
from jax.experimental import pallas as pl
from jax.experimental.pallas import tpu as pltpu
from jax.experimental.pallas import tpu_sc as plsc

NC, NS = 2, 16
NW = NC * NS
K = 8
NBUF = 6
LAG = 3


def _make_gather(n_ids: int, hidden: int):
  bpw = n_ids // NW
  nch = bpw // K
  main = (nch // NBUF) * NBUF
  mesh = plsc.VectorSubcoreMesh(core_axis_name="c", subcore_axis_name="s")

  @functools.partial(
      pl.kernel,
      mesh=mesh,
      out_type=jax.ShapeDtypeStruct((n_ids, hidden), jnp.float32),
      scratch_types=[
          pltpu.VMEM((bpw,), jnp.int32),
          [pltpu.VMEM((K, hidden), jnp.float32) for _ in range(NBUF)],
          [pltpu.SemaphoreType.DMA for _ in range(NBUF)],
          [pltpu.SemaphoreType.DMA for _ in range(NBUF)],
      ],
  )
  def gather(tbl_hbm, idx_hbm, out_hbm, idx_v, bufs, gsems, ssems):
    wid = lax.axis_index("s") * NC + lax.axis_index("c")
    base = pl.multiple_of(wid * bpw, 8)
    pltpu.sync_copy(idx_hbm.at[pl.ds(base, bpw)], idx_v)

    def g_desc(ch, j):
      off = pl.multiple_of(ch * K, 8)
      return pltpu.make_async_copy(
          tbl_hbm.at[idx_v.at[pl.ds(off, K)]], bufs[j], gsems[j])

    def s_desc(ch, j):
      off = pl.multiple_of(ch * K, 8)
      return pltpu.make_async_copy(
          bufs[j], out_hbm.at[pl.ds(base + off, K)], ssems[j])

    def visit(ch, j, static):
      g_desc(ch, j).wait()
      s_desc(ch, j).start()
      nx = ch + NBUF - LAG
      jn = (j + NBUF - LAG) % NBUF

      def follow_up():
        if static:
          if nx >= NBUF:
            s_desc(nx - NBUF, jn).wait()
          g_desc(nx, jn).start()
        else:
          @pl.when(nx >= NBUF)
          def _():
            s_desc(nx - NBUF, jn).wait()
          g_desc(nx, jn).start()

      if static:
        if nx < nch:
          follow_up()
      else:
        @pl.when(nx < nch)
        def _():
          follow_up()

    for j in range(NBUF - LAG):
      g_desc(j, j).start()

    @pl.loop(0, main, step=NBUF)
    def _(t):
      for j in range(NBUF):
        visit(t + j, j, static=False)

    for ch in range(main, nch):
      visit(ch, ch % NBUF, static=True)

    for c in range(nch - NBUF, nch):
      s_desc(c, c % NBUF).wait()

  return gather


@jax.jit
def kernel(input_ids, embed_table):
  b, s = input_ids.shape
  v, h = embed_table.shape
  ids = input_ids.reshape(b * s).astype(jnp.int32)
  out = _make_gather(b * s, h)(embed_table, ids)
  return out.reshape(b, s, h)


# speedup vs baseline: 1.2408x; 1.0037x over previous
import functools

import jax
import jax.numpy as jnp
from jax import lax
from jax.experimental import pallas as pl
from jax.experimental.pallas import tpu as pltpu
from jax.experimental.pallas import tpu_sc as plsc

NC, NS = 2, 16
NW = NC * NS
K = 8
NBUF = 4
NSP = 2
LAG = 2


def _make_gather(n_ids: int, hidden: int):
  bpw = n_ids // NW
  nch = bpw // K
  mesh = plsc.VectorSubcoreMesh(core_axis_name="c", subcore_axis_name="s")

  @functools.partial(
      pl.kernel,
      mesh=mesh,
      out_type=jax.ShapeDtypeStruct((n_ids, hidden), jnp.float32),
      scratch_types=[
          pltpu.VMEM((bpw,), jnp.int32),
          [pltpu.VMEM((K, hidden), jnp.float32) for _ in range(NBUF)],
          pltpu.VMEM_SHARED((NS, NSP, K, hidden), jnp.float32),
          [pltpu.SemaphoreType.DMA for _ in range(NBUF)],
          [pltpu.SemaphoreType.DMA for _ in range(NSP)],
          [pltpu.SemaphoreType.DMA for _ in range(NSP)],
      ],
  )
  def gather(tbl_hbm, idx_hbm, out_hbm, idx_v, bufs, spm, gsems, csems, ssems):
    cid = lax.axis_index("c")
    sid = lax.axis_index("s")
    wid = sid * NC + cid
    base = pl.multiple_of(wid * bpw, 8)
    pltpu.sync_copy(idx_hbm.at[pl.ds(base, bpw)], idx_v)

    def g_desc(ch, j):
      off = pl.multiple_of(ch * K, 8)
      return pltpu.make_async_copy(
          tbl_hbm.at[idx_v.at[pl.ds(off, K)]], bufs[j], gsems[j])

    def c_desc(j, p):
      return pltpu.make_async_copy(bufs[j], spm.at[sid, p], csems[p])

    def s_desc(ch, p):
      off = pl.multiple_of(ch * K, 8)
      return pltpu.make_async_copy(
          spm.at[sid, p], out_hbm.at[pl.ds(base + off, K)], ssems[p])

    for j in range(NBUF - LAG):
      g_desc(j, j).start()

    @pl.loop(0, nch, step=NBUF)
    def _(t):
      for j in range(NBUF):
        ch = t + j
        p = j % NSP
        g_desc(ch, j).wait()

        @pl.when(ch >= NSP)
        def _():
          s_desc(ch - NSP, p).wait()
        c_desc(j, p).start()
        c_desc(j, p).wait()
        s_desc(ch, p).start()
        jn = (j + NBUF - LAG) % NBUF

        @pl.when(ch + NBUF - LAG < nch)
        def _():
          g_desc(ch + NBUF - LAG, jn).start()

    for c in range(nch - NSP, nch):
      s_desc(c, c % NSP).wait()

  return gather


@jax.jit
def kernel(input_ids, embed_table):
  b, s = input_ids.shape
  v, h = embed_table.shape
  ids = input_ids.reshape(b * s).astype(jnp.int32)
  out = _make_gather(b * s, h)(embed_table, ids)
  return out.reshape(b, s, h)


# speedup vs baseline: 1.2465x; 1.0046x over previous
import functools

import jax
import jax.numpy as jnp
from jax import lax
from jax.experimental import pallas as pl
from jax.experimental.pallas import tpu as pltpu
from jax.experimental.pallas import tpu_sc as plsc

NC, NS = 2, 16
NW = NC * NS
K = 8
NBUF = 4
LAG = 2


def _make_gather(n_ids: int, hidden: int):
  bpw = n_ids // NW
  nch = bpw // K
  mesh = plsc.VectorSubcoreMesh(core_axis_name="c", subcore_axis_name="s")

  @functools.partial(
      pl.kernel,
      mesh=mesh,
      out_type=jax.ShapeDtypeStruct((n_ids, hidden), jnp.float32),
      scratch_types=[
          pltpu.VMEM((bpw,), jnp.int32),
          [pltpu.VMEM((K, hidden), jnp.float32) for _ in range(NBUF)],
          [pltpu.SemaphoreType.DMA for _ in range(NBUF)],
          [pltpu.SemaphoreType.DMA for _ in range(NBUF)],
      ],
  )
  def gather(tbl_hbm, idx_hbm, out_hbm, idx_v, bufs, gsems, ssems):
    wid = lax.axis_index("s") * NC + lax.axis_index("c")
    base = pl.multiple_of(wid * bpw, 8)
    pltpu.sync_copy(idx_hbm.at[pl.ds(base, bpw)], idx_v)

    def g_desc(ch, j):
      off = pl.multiple_of(ch * K, 8)
      return pltpu.make_async_copy(
          tbl_hbm.at[idx_v.at[pl.ds(off, K)]], bufs[j], gsems[j])

    def s_desc(ch, j):
      off = pl.multiple_of(ch * K, 8)
      return pltpu.make_async_copy(
          bufs[j], out_hbm.at[pl.ds(base + off, K)], ssems[j])

    for j in range(NBUF - LAG):
      g_desc(j, j).start()

    @pl.loop(0, nch, step=NBUF)
    def _(t):
      for j in range(NBUF):
        ch = t + j
        g_desc(ch, j).wait()
        s_desc(ch, j).start()
        jn = (j + NBUF - LAG) % NBUF

        @pl.when(ch + NBUF - LAG < nch)
        def _():
          @pl.when(ch + NBUF - LAG >= NBUF)
          def _():
            s_desc(ch - LAG, jn).wait()
          g_desc(ch + NBUF - LAG, jn).start()

    for c in range(nch - NBUF, nch):
      s_desc(c, c % NBUF).wait()

  return gather


@jax.jit
def kernel(input_ids, embed_table):
  b, s = input_ids.shape
  v, h = embed_table.shape
  ids = input_ids.reshape(b * s).astype(jnp.int32)
  out = _make_gather(b * s, h)(embed_table, ids)
  return out.reshape(b, s, h)
